# baseline (speedup 1.0000x reference)
import functools

import jax
import jax.numpy as jnp
from jax import lax
from jax.experimental import pallas as pl
from jax.experimental.pallas import tpu as pltpu
from jax.experimental.pallas import tpu_sc as plsc

D = 1024
S = 77
B = 1024
LANES = 16
NBR = 32
BLKS = B // NBR
NW = 32
M = S * BLKS // NW
NBUF = 3
SROWS = 80

_mesh = plsc.VectorSubcoreMesh(core_axis_name="c", subcore_axis_name="s")


@functools.partial(
    pl.kernel,
    out_type=jax.ShapeDtypeStruct((S, B, D), jnp.float32),
    mesh=_mesh,
    scratch_types=[
        [pltpu.VMEM((NBR, D), jnp.float32) for _ in range(NBUF)],
        pltpu.VMEM((4 * D,), jnp.float32),
        [pltpu.VMEM((NBR,), jnp.int32) for _ in range(NBUF)],
        [pltpu.SemaphoreType.DMA for _ in range(NBUF)],
        [pltpu.SemaphoreType.DMA for _ in range(NBUF)],
        [pltpu.SemaphoreType.DMA for _ in range(NBUF)],
    ],
)
def _emb_kernel(ids_hbm, tok_hbm, pos_hbm, out_hbm,
                rows_b, pos_all, idx_b, gsem, wsem, isem):
    wid = lax.axis_index("s") * 2 + lax.axis_index("c")
    base = wid * M
    s_lo = base // BLKS
    pltpu.sync_copy(pos_hbm.at[pl.ds(s_lo * D, 4 * D)], pos_all)

    def coords(u):
        t = base + u
        return t // BLKS, (t % BLKS) * NBR

    def idx_desc(u, k):
        s, b0 = coords(u)
        return pltpu.make_async_copy(
            ids_hbm.at[pl.ds(s * B + b0, NBR)], idx_b[k], isem[k])

    def gather_desc(k):
        return pltpu.make_async_copy(tok_hbm.at[idx_b[k]], rows_b[k], gsem[k])

    def write_desc(u, k):
        s, b0 = coords(u)
        return pltpu.make_async_copy(
            rows_b[k], out_hbm.at[s, pl.ds(b0, NBR)], wsem[k])

    def add_task(u, k):
        ref = rows_b[k]
        s, _ = coords(u)
        soff = (s - s_lo) * D
        for jg in range(D // (LANES * 16)):
            pvs = [pos_all[pl.ds(soff + (jg * 16 + jj) * LANES, LANES)]
                   for jj in range(16)]

            def rbody(r2, carry):
                for dr in range(2):
                    r = 2 * r2 + dr
                    for jj in range(16):
                        sl = pl.ds((jg * 16 + jj) * LANES, LANES)
                        ref[r, sl] = ref[r, sl] + pvs[jj]
                return carry

            lax.fori_loop(0, NBR // 2, rbody, 0)

    def slot_a(u, k, fill):
        if not fill:
            write_desc(u - NBUF, k).wait()
        idx_desc(u, k).wait()
        gather_desc(k).start()
        idx_desc(u + 1, (k + 1) % NBUF).start()

    def slot_b(u, k):
        gather_desc(k).wait()
        add_task(u, k)
        write_desc(u, k).start()

    idx_desc(0, 0).start()

    for u in range(NBUF):
        slot_a(u, u, True)
        if u >= 1:
            slot_b(u - 1, u - 1)

    def group_body(h, carry):
        u0 = 3 * h
        for dk in range(NBUF):
            slot_a(u0 + dk, dk, False)
            slot_b(u0 + dk - 1, (dk - 1) % NBUF)
        return carry

    lax.fori_loop(1, (M - 2) // NBUF, group_body, 0)

    for u in range(M - 2, M):
        slot_a(u, u % NBUF, False)
        slot_b(u - 1, (u - 1) % NBUF)
    slot_b(M - 1, (M - 1) % NBUF)
    idx_desc(M, (M % NBUF)).wait()
    for u in range(M - NBUF, M):
        write_desc(u, u % NBUF).wait()


def kernel(input_ids, token_embedding, position_embedding):
    ids_t = jnp.pad(input_ids.astype(jnp.int32).T,
                    ((0, SROWS - S), (0, 0))).reshape(-1)
    pos_f = jnp.pad(position_embedding, ((0, SROWS - S), (0, 0))).reshape(-1)
    out_t = _emb_kernel(ids_t, token_embedding, pos_f)
    return out_t.transpose(1, 0, 2)


# speedup vs baseline: 1.1832x; 1.1832x over previous
import functools

import jax
import jax.numpy as jnp
from jax import lax
from jax.experimental import pallas as pl
from jax.experimental.pallas import tpu as pltpu
from jax.experimental.pallas import tpu_sc as plsc

D = 1024
S = 77
B = 1024
LANES = 16
NBR = 32
BLKS = B // NBR
NW = 32
M = S * BLKS // NW
NBUF = 3
SROWS = 80

_mesh = plsc.VectorSubcoreMesh(core_axis_name="c", subcore_axis_name="s")


@functools.partial(
    pl.kernel,
    out_type=jax.ShapeDtypeStruct((S, B, D), jnp.float32),
    mesh=_mesh,
    scratch_types=[
        [pltpu.VMEM((NBR, D), jnp.float32) for _ in range(NBUF)],
        pltpu.VMEM((4 * D,), jnp.float32),
        [pltpu.VMEM((NBR,), jnp.int32) for _ in range(NBUF)],
        [pltpu.SemaphoreType.DMA for _ in range(NBUF)],
        [pltpu.SemaphoreType.DMA for _ in range(NBUF)],
        [pltpu.SemaphoreType.DMA for _ in range(NBUF)],
    ],
)
def _emb_kernel(ids_hbm, tok_hbm, pos_hbm, out_hbm,
                rows_b, pos_all, idx_b, gsem, wsem, isem):
    wid = lax.axis_index("s") * 2 + lax.axis_index("c")
    base = wid * M
    s_lo = base // BLKS
    pltpu.sync_copy(pos_hbm.at[pl.ds(s_lo * D, 4 * D)], pos_all)

    def coords(u):
        t = base + u
        return t // BLKS, (t % BLKS) * NBR

    def idx_desc(u, k):
        s, b0 = coords(u)
        return pltpu.make_async_copy(
            ids_hbm.at[pl.ds(s * B + b0, NBR)], idx_b[k], isem[k])

    def gather_desc(k):
        return pltpu.make_async_copy(tok_hbm.at[idx_b[k]], rows_b[k], gsem[k])

    def write_desc(u, k):
        s, b0 = coords(u)
        return pltpu.make_async_copy(
            rows_b[k], out_hbm.at[s, pl.ds(b0, NBR)], wsem[k])

    def add_task(u, k):
        ref = rows_b[k]
        s, _ = coords(u)
        soff = (s - s_lo) * D
        for jg in range(D // (LANES * 16)):
            pvs = [pos_all[pl.ds(soff + (jg * 16 + jj) * LANES, LANES)]
                   for jj in range(16)]

            def rbody(r, carry):
                for jj in range(16):
                    sl = pl.ds((jg * 16 + jj) * LANES, LANES)
                    ref[r, sl] = ref[r, sl] + pvs[jj]
                return carry

            lax.fori_loop(0, NBR, rbody, 0)

    def slot_a(u, k, fill):
        if not fill:
            write_desc(u - NBUF, k).wait()
        idx_desc(u, k).wait()
        gather_desc(k).start()
        idx_desc(u + 1, (k + 1) % NBUF).start()

    def slot_b(u, k):
        gather_desc(k).wait()
        add_task(u, k)
        write_desc(u, k).start()

    idx_desc(0, 0).start()

    for u in range(NBUF):
        slot_a(u, u, True)
        if u >= 1:
            slot_b(u - 1, u - 1)

    def group_body(h, carry):
        u0 = 3 * h
        for dk in range(NBUF):
            slot_a(u0 + dk, dk, False)
            slot_b(u0 + dk - 1, (dk - 1) % NBUF)
        return carry

    lax.fori_loop(1, (M - 2) // NBUF, group_body, 0)

    for u in range(M - 2, M):
        slot_a(u, u % NBUF, False)
        slot_b(u - 1, (u - 1) % NBUF)
    slot_b(M - 1, (M - 1) % NBUF)
    idx_desc(M, (M % NBUF)).wait()
    for u in range(M - NBUF, M):
        write_desc(u, u % NBUF).wait()


def kernel(input_ids, token_embedding, position_embedding):
    ids_t = jnp.pad(input_ids.astype(jnp.int32).T,
                    ((0, SROWS - S), (0, 0))).reshape(-1)
    pos_f = jnp.pad(position_embedding, ((0, SROWS - S), (0, 0))).reshape(-1)
    out_t = _emb_kernel(ids_t, token_embedding, pos_f)
    return out_t.transpose(1, 0, 2)
